# baseline (speedup 1.0000x reference)
import functools

import jax
import jax.numpy as jnp
from jax import lax
from jax.experimental import pallas as pl
from jax.experimental.pallas import tpu as pltpu
from jax.experimental.pallas import tpu_sc as plsc

VOCAB = 1000000
SEQ_LEN = 200
D_MODEL = 64
BATCH = 4096
BS = 2
NGB = 2
DUNROLL = 4
NPAIR = SEQ_LEN // BS


def _sc_call(idxT, tab2, pos_enc):
    info = plsc.get_sparse_core_info()
    nc, ns = info.num_cores, info.num_subcores
    nw = nc * ns
    bcol = BATCH // nw
    ncc = bcol // 16

    mesh = plsc.VectorSubcoreMesh(core_axis_name="c", subcore_axis_name="s")

    scratch = (
        [pltpu.VMEM((SEQ_LEN, bcol), jnp.int32)]
        + [pltpu.VMEM((BS * bcol,), jnp.int32) for _ in range(NGB)]
        + [pltpu.VMEM((BS * bcol, 2 * D_MODEL), jnp.float32)
           for _ in range(NGB)]
        + [pltpu.VMEM((D_MODEL, bcol), jnp.float32)]
        + [pltpu.VMEM((SEQ_LEN, D_MODEL), jnp.float32)]
        + [pltpu.SemaphoreType.DMA for _ in range(NGB + 1)]
    )

    @functools.partial(
        pl.kernel,
        out_type=jax.ShapeDtypeStruct((SEQ_LEN, D_MODEL, BATCH), jnp.float32),
        mesh=mesh,
        scratch_types=scratch,
        compiler_params=pltpu.CompilerParams(
            use_tc_tiling_on_sc=True, needs_layout_passes=False),
    )
    def k(idxT_hbm, tab2_hbm, pos_hbm, out_hbm, idx_v, *rest):
        grp = rest[:NGB]
        gath = rest[NGB:2 * NGB]
        outt = rest[2 * NGB]
        pos_v = rest[2 * NGB + 1]
        gsem = rest[2 * NGB + 2:2 * NGB + 2 + NGB]
        osem = rest[2 * NGB + 2 + NGB]

        wid = lax.axis_index("s") * nc + lax.axis_index("c")
        b0 = pl.multiple_of(wid * bcol, bcol)

        pltpu.sync_copy(idxT_hbm.at[:, pl.ds(b0, bcol)], idx_v)
        pltpu.sync_copy(pos_hbm, pos_v)

        def issue_gather(sp, p):
            for h in range(BS):
                for cc in range(ncc):
                    v = idx_v[sp * BS + h, pl.ds(cc * 16, 16)]
                    grp[p][pl.ds(h * bcol + cc * 16, 16)] = (
                        lax.shift_right_logical(v, 1))
            pltpu.async_copy(tab2_hbm.at[grp[p]], gath[p], gsem[p])

        def wait_gather(p):
            pltpu.make_async_copy(tab2_hbm.at[grp[p]], gath[p], gsem[p]).wait()

        def issue_out(s):
            pltpu.async_copy(outt, out_hbm.at[s, :, pl.ds(b0, bcol)], osem)

        def wait_out(s):
            pltpu.make_async_copy(outt, out_hbm.at[s, :, pl.ds(b0, bcol)],
                                  osem).wait()

        def compute(s, p, h):
            offs, kvecs = [], []
            for cc in range(ncc):
                v = idx_v[s, pl.ds(cc * 16, 16)]
                offs.append(lax.shift_left(lax.bitwise_and(v, 1), 6))
                kvecs.append(lax.iota(jnp.int32, 16) + (h * bcol + cc * 16))
            sbc = lax.broadcast(s, (16,))

            def dbody(t, carry):
                offs_c, kvecs_c = carry
                d0 = t * DUNROLL
                pvs = [
                    plsc.load_gather(
                        pos_v, [sbc, lax.broadcast(d0 + u, (16,))])
                    for u in range(DUNROLL)
                ]
                for cc in range(ncc):
                    for u in range(DUNROLL):
                        vals = plsc.load_gather(
                            gath[p], [kvecs_c[cc], offs_c[cc] + (d0 + u)])
                        outt[d0 + u, pl.ds(cc * 16, 16)] = vals + pvs[u]
                return carry
            lax.fori_loop(0, D_MODEL // DUNROLL, dbody,
                          (tuple(offs), tuple(kvecs)))

        def step(sp, p, wait_o, issue_g):
            wait_gather(p)
            for h in range(BS):
                s = sp * BS + h
                if wait_o or h > 0:
                    wait_out(s - 1)
                compute(s, p, h)
                issue_out(s)
            if issue_g:
                issue_gather(sp + NGB, p)

        for p in range(NGB):
            issue_gather(p, p)
        for sp in range(NGB):
            step(sp, sp % NGB, wait_o=(sp > 0), issue_g=True)

        def sbody(blk, carry):
            sp0 = blk * NGB
            for j in range(NGB):
                step(sp0 + j, j, wait_o=True, issue_g=True)
            return carry
        lax.fori_loop(1, NPAIR // NGB - 1, sbody, 0)

        for j in range(NGB):
            sp = NPAIR - NGB + j
            step(sp, sp % NGB, wait_o=True, issue_g=False)
        wait_out(SEQ_LEN - 1)

    return k(idxT, tab2, pos_enc)


def kernel(indices, table, pos_enc):
    idxT = indices.T.astype(jnp.int32)
    tab2 = table.reshape(VOCAB // 2, 2 * D_MODEL)
    out = _sc_call(idxT, tab2, pos_enc)
    return out.transpose(2, 0, 1)


# speedup vs baseline: 1.4064x; 1.4064x over previous
import functools

import jax
import jax.numpy as jnp
from jax import lax
from jax.experimental import pallas as pl
from jax.experimental.pallas import tpu as pltpu
from jax.experimental.pallas import tpu_sc as plsc

VOCAB = 1000000
SEQ_LEN = 200
D_MODEL = 64
BATCH = 4096


def _sc_call(idx_flat, table, pos_enc):
    info = plsc.get_sparse_core_info()
    nc, ns = info.num_cores, info.num_subcores
    nw = nc * ns
    seqs_per_w = BATCH // nw
    rows_per_w = seqs_per_w * SEQ_LEN

    mesh = plsc.VectorSubcoreMesh(core_axis_name="c", subcore_axis_name="s")

    @functools.partial(
        pl.kernel,
        out_type=jax.ShapeDtypeStruct((BATCH * SEQ_LEN, D_MODEL), jnp.float32),
        mesh=mesh,
        scratch_types=[
            pltpu.VMEM((rows_per_w,), jnp.int32),
            pltpu.VMEM((SEQ_LEN, D_MODEL), jnp.float32),
            pltpu.VMEM((SEQ_LEN, D_MODEL), jnp.float32),
            pltpu.SemaphoreType.DMA,
        ],
        compiler_params=pltpu.CompilerParams(use_tc_tiling_on_sc=False),
    )
    def k(idx_hbm, table_hbm, pos_hbm, out_hbm, idx_v, pos_v, row_v, gsem):
        wid = lax.axis_index("s") * nc + lax.axis_index("c")
        base = pl.multiple_of(wid * rows_per_w, rows_per_w)
        pltpu.sync_copy(idx_hbm.at[pl.ds(base, rows_per_w)], idx_v)
        pltpu.sync_copy(pos_hbm, pos_v)

        def seq_body(s, carry):
            off = pl.multiple_of(s * SEQ_LEN, SEQ_LEN)
            g1 = pltpu.async_copy(
                table_hbm.at[idx_v.at[pl.ds(off, 128)]],
                row_v.at[pl.ds(0, 128)], gsem)
            g2 = pltpu.async_copy(
                table_hbm.at[idx_v.at[pl.ds(off + 128, SEQ_LEN - 128)]],
                row_v.at[pl.ds(128, SEQ_LEN - 128)], gsem)
            g1.wait()
            g2.wait()

            def add_body(i, c):
                for j in range(D_MODEL // 16):
                    plsc.addupdate(row_v.at[i, pl.ds(j * 16, 16)],
                                   pos_v[i, pl.ds(j * 16, 16)])
                return c
            lax.fori_loop(0, SEQ_LEN, add_body, 0, unroll=2)

            pltpu.sync_copy(row_v, out_hbm.at[pl.ds(base + off, SEQ_LEN)])
            return carry

        lax.fori_loop(0, seqs_per_w, seq_body, 0)

    return k(idx_flat, table, pos_enc)


def kernel(indices, table, pos_enc):
    idx_flat = indices.reshape(-1).astype(jnp.int32)
    out = _sc_call(idx_flat, table, pos_enc)
    return out.reshape(BATCH, SEQ_LEN, D_MODEL)
